# baseline (speedup 1.0000x reference)
import jax
import jax.numpy as jnp
from jax import lax
from jax.experimental import pallas as pl
from jax.experimental.pallas import tpu as pltpu
from jax.experimental.pallas import tpu_sc as plsc

VOCAB = 1000000
D_EMB = 64
D_MODEL = 128
B, S = 1024, 200
N_TOK = B * S

_INFO = plsc.get_sparse_core_info()
NC, NS = _INFO.num_cores, _INFO.num_subcores
NW = NC * NS
ROWS_PER_W = N_TOK // NW
CHUNK = 128
N_CHUNKS = ROWS_PER_W // CHUNK
IDX_ROWS = 56
NBUF = 4


def _gather_body(idx_hbm, table2_hbm, out_hbm, idx_v, rows_v, gsem, wsem):
    wid = lax.axis_index("s") * NC + lax.axis_index("c")
    base = wid * ROWS_PER_W
    pltpu.sync_copy(idx_hbm.at[wid], idx_v)

    def start_gather(c, slot):
        pltpu.async_copy(table2_hbm.at[idx_v.at[c]], rows_v.at[slot], gsem)

    def wait_gather(slot):
        pltpu.make_async_copy(
            table2_hbm.at[pl.ds(0, CHUNK)], rows_v.at[slot], gsem).wait()

    for p in range(NBUF - 1):
        start_gather(p, p)

    def step(c, _):
        slot = lax.rem(c, NBUF)

        @pl.when(c + NBUF - 1 < N_CHUNKS)
        def _():
            start_gather(c + NBUF - 1, lax.rem(c + NBUF - 1, NBUF))

        wait_gather(slot)
        pltpu.async_copy(rows_v.at[slot],
                         out_hbm.at[pl.ds(base + c * CHUNK, CHUNK)], wsem)

        @pl.when(c >= NBUF - 1)
        def _():
            pltpu.make_async_copy(
                table2_hbm.at[pl.ds(0, CHUNK)],
                out_hbm.at[pl.ds(0, CHUNK)], wsem).wait()

        return 0

    lax.fori_loop(0, N_CHUNKS, step, 0)
    pltpu.make_async_copy(
        table2_hbm.at[pl.ds(0, (NBUF - 1) * CHUNK)],
        out_hbm.at[pl.ds(0, (NBUF - 1) * CHUNK)], wsem).wait()


def _sc_gather(idxp, table2):
    mesh = plsc.VectorSubcoreMesh(core_axis_name="c", subcore_axis_name="s")
    k = pl.kernel(
        _gather_body,
        mesh=mesh,
        out_type=jax.ShapeDtypeStruct((N_TOK, 2 * D_EMB), jnp.float32),
        scratch_types=[
            pltpu.VMEM((IDX_ROWS, CHUNK), jnp.int32),
            pltpu.VMEM((NBUF, CHUNK, 2 * D_EMB), jnp.float32),
            pltpu.SemaphoreType.DMA,
            pltpu.SemaphoreType.DMA,
        ],
    )
    return k(idxp, table2)


CB = 32768
NGROUP = CB // 256
PACK_GRID = -(-VOCAB // CB)
T2_ROWS = PACK_GRID * (CB // 2)


def _pack_body(tt_ref, out_ref):
    for g in range(NGROUP):
        a = tt_ref[:, g * 256:g * 256 + 128].T
        bb_ = tt_ref[:, g * 256 + 128:(g + 1) * 256].T
        out_ref[g * 128:(g + 1) * 128, 0:D_EMB] = a
        out_ref[g * 128:(g + 1) * 128, D_EMB:2 * D_EMB] = bb_


def _pack_table(token_table_t):
    return pl.pallas_call(
        _pack_body,
        grid=(PACK_GRID,),
        in_specs=[pl.BlockSpec((D_EMB, CB), lambda i: (0, i))],
        out_specs=pl.BlockSpec((CB // 2, 2 * D_EMB), lambda i: (i, 0)),
        out_shape=jax.ShapeDtypeStruct((T2_ROWS, 2 * D_EMB), jnp.float32),
    )(token_table_t)


BB = 64


def _tc_body(tok2_ref, seq_ref, W2_ref, pos_ref, pos_W_ref, seg_ref,
             seg_W_ref, cb_ref, gamma_ref, beta_ref, out_ref):
    c = (jnp.dot(pos_ref[:], pos_W_ref[:],
                 preferred_element_type=jnp.float32)
         + jnp.dot(seg_ref[:], seg_W_ref[:],
                   preferred_element_type=jnp.float32)
         + cb_ref[:][None, :])
    y2 = jnp.dot(tok2_ref[:], W2_ref[:],
                 preferred_element_type=jnp.float32)
    y_lo = lax.slice(y2, (0, 0), (BB * S, D_MODEL)).reshape(BB, S, D_MODEL)
    y_hi = lax.slice(y2, (0, D_MODEL),
                     (BB * S, 2 * D_MODEL)).reshape(BB, S, D_MODEL)
    seq3 = seq_ref[:][:, :, None]
    y = jnp.where(((seq3 >> 7) & 1) == 1, y_hi, y_lo)
    y = y + c[None, :, :]
    mu = jnp.mean(y, axis=-1, keepdims=True)
    d = y - mu
    var = jnp.mean(d * d, axis=-1, keepdims=True)
    out_ref[:] = d * lax.rsqrt(var + 1e-5) * gamma_ref[:] + beta_ref[:]


def _tc_compute(tok2, seq, W2, pos_seq, pos_W, seg_row, seg_W, cb,
                gamma, beta):
    grid = (B // BB,)
    rep2 = lambda shape: pl.BlockSpec(shape, lambda i: (0, 0))
    rep1 = lambda shape: pl.BlockSpec(shape, lambda i: (0,))
    return pl.pallas_call(
        _tc_body,
        grid=grid,
        in_specs=[
            pl.BlockSpec((BB * S, 2 * D_EMB), lambda i: (i, 0)),
            pl.BlockSpec((BB, S), lambda i: (i, 0)),
            rep2((2 * D_EMB, 2 * D_MODEL)),
            rep2((S, D_EMB)),
            rep2((D_EMB, D_MODEL)),
            rep2((1, D_EMB)),
            rep2((D_EMB, D_MODEL)),
            rep1((D_MODEL,)),
            rep1((D_MODEL,)),
            rep1((D_MODEL,)),
        ],
        out_specs=pl.BlockSpec((BB, S, D_MODEL), lambda i: (i, 0, 0)),
        out_shape=jax.ShapeDtypeStruct((B, S, D_MODEL), jnp.float32),
    )(tok2, seq, W2, pos_seq, pos_W, seg_row, seg_W, cb, gamma, beta)


def kernel(token_table, W, b, pos_table, pos_W, pos_b, seg_table, seg_W,
           seg_b, gamma, beta, sequence):
    seq = sequence.astype(jnp.int32)
    table2 = _pack_table(token_table.T)
    idx3 = (((seq >> 8) << 7) | (seq & 127)).reshape(NW, N_CHUNKS, CHUNK)
    idxp = jnp.pad(idx3, ((0, 0), (0, IDX_ROWS - N_CHUNKS), (0, 0)))
    tok2 = _sc_gather(idxp, table2)
    zero = jnp.zeros((D_EMB, D_MODEL), jnp.float32)
    W2 = jnp.block([[W, zero], [zero, W]])
    cb = b + pos_b + seg_b
    return _tc_compute(tok2, seq, W2, pos_table[:S], pos_W, seg_table[0:1],
                       seg_W, cb, gamma, beta)
